# Initial kernel scaffold; baseline (speedup 1.0000x reference)
import jax
import jax.numpy as jnp
from jax.experimental import pallas as pl


def kernel(input_ids, seg_ids, word_embed, pos_embed, seg_embed):
    raise NotImplementedError("write your pallas kernel here")


# speedup vs baseline: 2.1250x; 2.1250x over previous
import functools

import jax
import jax.numpy as jnp
from jax import lax
from jax.experimental import pallas as pl
from jax.experimental.pallas import tpu as pltpu
from jax.experimental.pallas import tpu_sc as plsc

_VOCAB = 1000000
_EMBED = 64
_MAXLEN = 64
_SEGN = 2
_BATCH = 4096
_SEQ = 64

_NC = 2
_NS = 16
_NW = _NC * _NS
_ROWS = _BATCH * _SEQ
_RPW = _ROWS // _NW
_C = 1024
_NCHUNK = _RPW // _C
_G = 128
_NG = _C // _G


def _body(ids_hbm, seg_hbm, word_hbm, pos_hbm, segtab_hbm, out_hbm,
          idx_v, seg_v, rows_v, pos_v, segtab_v, gsem):
  wid = lax.axis_index("c") * _NS + lax.axis_index("s")
  base = wid * _RPW

  pltpu.sync_copy(pos_hbm, pos_v)
  pltpu.sync_copy(segtab_hbm, segtab_v)

  def chunk_body(c, _):
    cbase = pl.multiple_of(base + c * _C, _C)
    pltpu.sync_copy(ids_hbm.at[pl.ds(pl.multiple_of(cbase // _G, _NG), _NG)],
                    idx_v)
    pltpu.sync_copy(seg_hbm.at[pl.ds(cbase, _C)], seg_v)
    cps = []
    for j in range(_NG):
      cps.append(pltpu.async_copy(word_hbm.at[idx_v.at[j]],
                                  rows_v.at[pl.ds(j * _G, _G)], gsem))
    for cp in cps:
      cp.wait()

    def grp_body(t, _):
      rbase = t * 16
      seg16 = seg_v[pl.ds(rbase, 16)]
      for i in range(16):
        r = rbase + i
        s = lax.rem(r, _MAXLEN)
        g = seg16[i]
        for jj in range(_EMBED // 16):
          d = pl.ds(jj * 16, 16)
          rows_v[r, d] = rows_v[r, d] + pos_v[s, d] + segtab_v[g, d]
      return _
    lax.fori_loop(0, _C // 16, grp_body, None)

    pltpu.sync_copy(rows_v, out_hbm.at[pl.ds(cbase, _C)])
    return _

  lax.fori_loop(0, _NCHUNK, chunk_body, None)


@functools.partial(
    pl.kernel,
    out_type=jax.ShapeDtypeStruct((_ROWS, _EMBED), jnp.float32),
    mesh=plsc.VectorSubcoreMesh(core_axis_name="c", subcore_axis_name="s"),
    scratch_types=[
        pltpu.VMEM((_NG, _G), jnp.int32),
        pltpu.VMEM((_C,), jnp.int32),
        pltpu.VMEM((_C, _EMBED), jnp.float32),
        pltpu.VMEM((_MAXLEN, _EMBED), jnp.float32),
        pltpu.VMEM((_SEGN, _EMBED), jnp.float32),
        pltpu.SemaphoreType.DMA,
    ],
    compiler_params=pltpu.CompilerParams(use_tc_tiling_on_sc=False),
)
def _embed_sc(*refs):
  _body(*refs)


@jax.jit
def kernel(input_ids, seg_ids, word_embed, pos_embed, seg_embed):
  ids2d = input_ids.astype(jnp.int32).reshape(_ROWS // _G, _G)
  segf = seg_ids.astype(jnp.int32).reshape(_ROWS)
  out = _embed_sc(ids2d, segf, word_embed, pos_embed, seg_embed)
  return out.reshape(_BATCH, _SEQ, _EMBED)
